# baseline (speedup 1.0000x reference)
import functools

import jax
import jax.numpy as jnp
from jax import lax
from jax.experimental import pallas as pl
from jax.experimental.pallas import tpu as pltpu
from jax.experimental.pallas import tpu_sc as plsc

N_NODES = 10000
N_EDGES = 320000
D = 128
N_LAYERS = 3
N_TYPES = 8

NC = 2
NSUB = 16
NW = NC * NSUB
EPW = N_EDGES // NW
CHUNK = 50
NCH = EPW // CHUNK
NPAD = 10112
ROWS_PER_TILE = NPAD // NSUB
DW = D // 2

_f32 = jnp.float32


def _sc_segsum_body(gidx_hbm, sidx_hbm, table_hbm, out_hbm,
                    gidx_v, sidx_v, bf0, bf1, f0, f1, accum,
                    g0, g1, s0, s1):
    cid = lax.axis_index("c")
    sid = lax.axis_index("s")
    wid = cid * NSUB + sid
    bfbufs = (bf0, bf1)
    fbufs = (f0, f1)
    gsems = (g0, g1)
    ssems = (s0, s1)

    def _zrow(i, carry):
        for j in range(D // 16):
            f0[i, pl.ds(j * 16, 16)] = jnp.zeros((16,), _f32)
        return carry
    lax.fori_loop(0, CHUNK, _zrow, 0)
    row_base = sid * ROWS_PER_TILE
    for off, n in ((0, 48), (48, 48), (96, 48), (144, 48), (192, 48),
                   (240, 48), (288, 48), (336, 48), (384, 48), (432, 48),
                   (480, 48), (528, 48), (576, 48), (624, 8)):
        pltpu.sync_copy(f0.at[pl.ds(0, n)],
                        accum.at[pl.ds(row_base + off, n)])
    plsc.subcore_barrier()

    pltpu.sync_copy(gidx_hbm.at[pl.ds(wid * NCH, NCH)], gidx_v)
    pltpu.sync_copy(sidx_hbm.at[pl.ds(wid * NCH, NCH)], sidx_v)

    def _gather_start(c, b):
        pltpu.async_copy(table_hbm.at[gidx_v.at[c]], bfbufs[b], gsems[b])

    def _gather_wait(c, b):
        pltpu.make_async_copy(table_hbm.at[gidx_v.at[c]], bfbufs[b],
                              gsems[b]).wait()

    def _scatter_start(c, b):
        pltpu.async_copy(fbufs[b], accum.at[sidx_v.at[c]], ssems[b], add=True)

    def _scatter_wait(c, b):
        pltpu.make_async_copy(fbufs[b], accum.at[sidx_v.at[c]],
                              ssems[b]).wait()

    def _convert(bfb, fb):
        def _row(r, carry):
            for k in range(DW // 16):
                w = bfb[r, pl.ds(k * 16, 16)]
                wb = plsc.bitcast(w, jnp.bfloat16)
                lo, hi = plsc.unpack(wb, format=plsc.PackFormat.INTERLEAVED)
                fb[r, pl.ds(k * 16, 16)] = lo
                fb[r, pl.ds(DW + k * 16, 16)] = hi
            return carry
        lax.fori_loop(0, CHUNK, _row, 0)

    def _steady(c, b, first=False, last=False):
        _gather_wait(c, b)
        if not first:
            _scatter_wait(c - 2, b)
        _convert(bfbufs[b], fbufs[b])
        _scatter_start(c, b)
        if not last:
            _gather_start(c + 2, b)

    _gather_start(0, 0)
    _gather_start(1, 1)
    _steady(0, 0, first=True)
    _steady(1, 1, first=True)

    def _pair(q, carry):
        c0 = q * 2
        for b in range(2):
            _steady(c0 + b, b)
        return carry
    lax.fori_loop(1, NCH // 2 - 1, _pair, 0)

    _steady(NCH - 2, 0, last=True)
    _steady(NCH - 1, 1, last=True)
    _scatter_wait(NCH - 2, 0)
    _scatter_wait(NCH - 1, 1)

    plsc.subcore_barrier()
    pltpu.sync_copy(
        accum.at[pl.ds(row_base, ROWS_PER_TILE)],
        out_hbm.at[cid, pl.ds(row_base, ROWS_PER_TILE)])


@functools.cache
def _make_sc_segsum():
    return pl.kernel(
        _sc_segsum_body,
        out_type=jax.ShapeDtypeStruct((NC, NPAD, D), _f32),
        mesh=plsc.VectorSubcoreMesh(
            core_axis_name="c", subcore_axis_name="s",
            num_cores=NC, num_subcores=NSUB),
        compiler_params=pltpu.CompilerParams(use_tc_tiling_on_sc=False,
                                            needs_layout_passes=False),
        scratch_types=[
            pltpu.VMEM((NCH, CHUNK), jnp.int32),
            pltpu.VMEM((NCH, CHUNK), jnp.int32),
            pltpu.VMEM((CHUNK, DW), _f32),
            pltpu.VMEM((CHUNK, DW), _f32),
            pltpu.VMEM((CHUNK, D), _f32),
            pltpu.VMEM((CHUNK, D), _f32),
            pltpu.VMEM_SHARED((NPAD, D), _f32),
        ] + [pltpu.SemaphoreType.DMA] * 4,
    )


def _sc_segsum(gidx, sidx, table):
    return _make_sc_segsum()(gidx, sidx, table)


BR = 1000
GRID = N_NODES // BR


def _pack_cols(p):
    lo = jax.lax.bitcast_convert_type(p[:, :DW].astype(jnp.bfloat16),
                                      jnp.uint16).astype(jnp.uint32)
    hi = jax.lax.bitcast_convert_type(p[:, DW:].astype(jnp.bfloat16),
                                      jnp.uint16).astype(jnp.uint32)
    return jax.lax.bitcast_convert_type(lo | (hi << 16), _f32)


def _embed_body(x_ref, t_ref, params_ref, Wsp_ref, bsp_ref, tt_ref,
                Wpp_ref, bpp_ref, W0_ref, hs_ref, hr_ref, ps_ref):
    hs = jnp.maximum(jnp.log1p(x_ref[...]) * Wsp_ref[...] + bsp_ref[...], 0.0)
    hs_ref[...] = hs
    ps_ref[...] = _pack_cols(jnp.dot(hs, W0_ref[...], preferred_element_type=_f32))
    onehot = (t_ref[...] == lax.broadcasted_iota(jnp.int32, (BR, N_TYPES), 1)
              ).astype(_f32)
    hr = (jnp.dot(onehot, tt_ref[...], preferred_element_type=_f32)
          + jnp.dot(params_ref[...], Wpp_ref[...], preferred_element_type=_f32)
          + bpp_ref[...])
    hr_ref[...] = jnp.maximum(hr, 0.0)


def _embed(x2, t2, params, Wsp, bsp2, tt, Wpp, bpp2, W0):
    full = lambda shape: pl.BlockSpec(shape, lambda i: (0,) * len(shape))
    rows = lambda w: pl.BlockSpec((BR, w), lambda i: (i, 0))
    return pl.pallas_call(
        _embed_body,
        grid=(GRID,),
        in_specs=[rows(1), rows(1), rows(4), full((1, D)), full((1, D)),
                  full((N_TYPES, D)), full((4, D)), full((1, D)),
                  full((D, D))],
        out_specs=[rows(D), rows(D), rows(DW)],
        out_shape=[jax.ShapeDtypeStruct((N_NODES, D), _f32)] * 2
        + [jax.ShapeDtypeStruct((N_NODES, DW), _f32)],
    )(x2, t2, params, Wsp, bsp2, tt, Wpp, bpp2, W0)


def _update_body(with_p, with_sum, *refs):
    it = iter(refs)
    h_ref, p0_ref, p1_ref = next(it), next(it), next(it)
    W_ref = next(it) if with_p else None
    b_ref = next(it)
    out_ref = next(it)
    pn_ref = next(it) if with_p else None
    sum_ref = next(it) if with_sum else None
    h = jnp.maximum(h_ref[...] + p0_ref[0] + p1_ref[0] + b_ref[...], 0.0)
    out_ref[...] = h
    if with_p:
        pn_ref[...] = _pack_cols(jnp.dot(h, W_ref[...], preferred_element_type=_f32))
    if with_sum:
        @pl.when(pl.program_id(0) == 0)
        def _():
            sum_ref[...] = jnp.zeros_like(sum_ref)
        sum_ref[...] += jnp.sum(h, axis=0, keepdims=True)


def _update(h, parts, W_next, b2, with_sum):
    with_p = W_next is not None
    rows = pl.BlockSpec((BR, D), lambda i: (i, 0))
    full = lambda shape: pl.BlockSpec(shape, lambda i: (0, 0))
    in_specs = [rows,
                pl.BlockSpec((1, BR, D), lambda i: (0, i, 0)),
                pl.BlockSpec((1, BR, D), lambda i: (1, i, 0))]
    args = [h, parts, parts]
    if with_p:
        in_specs.append(full((D, D)))
        args.append(W_next)
    in_specs.append(full((1, D)))
    args.append(b2)
    out_specs = [rows]
    out_shape = [jax.ShapeDtypeStruct((N_NODES, D), _f32)]
    if with_p:
        out_specs.append(pl.BlockSpec((BR, DW), lambda i: (i, 0)))
        out_shape.append(jax.ShapeDtypeStruct((N_NODES, DW), _f32))
    if with_sum:
        out_specs.append(full((1, D)))
        out_shape.append(jax.ShapeDtypeStruct((1, D), _f32))
    res = pl.pallas_call(
        functools.partial(_update_body, with_p, with_sum),
        grid=(GRID,),
        in_specs=in_specs,
        out_specs=out_specs,
        out_shape=out_shape,
    )(*args)
    return res


def kernel(initial_state, propensity_params, W_sp, b_sp, type_table, W_pp,
           b_pp, W_s2r, b_s2r, W_r2s, b_r2s, propensity_types, edge_index):
    x2 = initial_state.reshape(N_NODES, 1)
    t2 = propensity_types.reshape(N_NODES, 1)
    src = edge_index[0].reshape(NW * NCH, CHUNK)
    dst = edge_index[1].reshape(NW * NCH, CHUNK)
    bsp2 = b_sp.reshape(1, D)
    bpp2 = b_pp.reshape(1, D)

    h_s, h_r, p_s = _embed(x2, t2, propensity_params, W_sp, bsp2, type_table,
                           W_pp, bpp2, W_s2r[0])

    sum_r = sum_s = None
    for l in range(N_LAYERS):
        last = l == N_LAYERS - 1
        parts = _sc_segsum(src, dst, p_s)
        res = _update(h_r, parts, W_r2s[l], b_s2r[l].reshape(1, D), last)
        if last:
            h_r, p_r, sum_r = res
        else:
            h_r, p_r = res
        parts = _sc_segsum(dst, src, p_r)
        res = _update(h_s, parts,
                      None if last else W_s2r[l + 1],
                      b_r2s[l].reshape(1, D), last)
        if last:
            h_s, sum_s = res
        else:
            h_s, p_s = res

    context = jnp.concatenate(
        [sum_s[0] * (1.0 / N_NODES), sum_r[0] * (1.0 / N_NODES)], axis=-1)
    return (h_s, h_r, context)


# speedup vs baseline: 1.7237x; 1.7237x over previous
import functools

import jax
import jax.numpy as jnp
from jax import lax
from jax.experimental import pallas as pl
from jax.experimental.pallas import tpu as pltpu
from jax.experimental.pallas import tpu_sc as plsc

N_NODES = 10000
N_EDGES = 320000
D = 128
N_LAYERS = 3
N_TYPES = 8

NC = 2
NSUB = 16
NW = NC * NSUB
EPW = N_EDGES // NW
CHUNK = 125
NCH = EPW // CHUNK
NPAD = 10112
ROWS_PER_TILE = NPAD // NSUB
PCH = 40

_f32 = jnp.float32


def _sc_segsum_body(gidx_hbm, sidx_hbm, table_hbm, out_hbm,
                    gidx_v, sidx_v, rows0, rows1, g0, g1, s0, s1, accum):
    cid = lax.axis_index("c")
    sid = lax.axis_index("s")
    wid = cid * NSUB + sid
    bufs = (rows0, rows1)
    gsems = (g0, g1)
    ssems = (s0, s1)

    def _zrow(i, carry):
        for j in range(D // 16):
            rows0[i, pl.ds(j * 16, 16)] = jnp.zeros((16,), _f32)
        return carry
    lax.fori_loop(0, CHUNK, _zrow, 0)
    row_base = sid * ROWS_PER_TILE
    for off, n in ((0, 120), (120, 120), (240, 120), (360, 120), (480, 120),
                   (600, 32)):
        pltpu.sync_copy(rows0.at[pl.ds(0, n)],
                        accum.at[pl.ds(row_base + off, n)])
    plsc.subcore_barrier()

    def _gather_start(c, b):
        pltpu.async_copy(table_hbm.at[gidx_v.at[c]], bufs[b], gsems[b])

    def _gather_wait(c, b):
        pltpu.make_async_copy(table_hbm.at[gidx_v.at[c]], bufs[b],
                              gsems[b]).wait()

    def _scatter_start(c, b):
        pltpu.async_copy(bufs[b], accum.at[sidx_v.at[c]], ssems[b], add=True)

    def _scatter_wait(c, b):
        pltpu.make_async_copy(bufs[b], accum.at[sidx_v.at[c]],
                              ssems[b]).wait()

    def _step(c, b, first=False, last=False):
        _gather_wait(c, b)
        if not first:
            _scatter_wait(c - 2, b)
        _scatter_start(c, b)
        if not last:
            _gather_start(c + 2, b)

    for p in range(NCH // PCH):
        pltpu.sync_copy(gidx_hbm.at[pl.ds(wid * NCH + p * PCH, PCH)], gidx_v)
        pltpu.sync_copy(sidx_hbm.at[pl.ds(wid * NCH + p * PCH, PCH)], sidx_v)

        _gather_start(0, 0)
        _gather_start(1, 1)
        _step(0, 0, first=True)
        _step(1, 1, first=True)

        def _pair(q, carry):
            c0 = q * 2
            for b in range(2):
                _step(c0 + b, b)
            return carry
        lax.fori_loop(1, PCH // 2 - 1, _pair, 0)

        _step(PCH - 2, 0, last=True)
        _step(PCH - 1, 1, last=True)
        _scatter_wait(PCH - 2, 0)
        _scatter_wait(PCH - 1, 1)

    plsc.subcore_barrier()
    pltpu.sync_copy(
        accum.at[pl.ds(row_base, ROWS_PER_TILE)],
        out_hbm.at[cid, pl.ds(row_base, ROWS_PER_TILE)])


@functools.cache
def _make_sc_segsum():
    return pl.kernel(
        _sc_segsum_body,
        out_type=jax.ShapeDtypeStruct((NC, NPAD, D), _f32),
        mesh=plsc.VectorSubcoreMesh(
            core_axis_name="c", subcore_axis_name="s",
            num_cores=NC, num_subcores=NSUB),
        scratch_types=[
            pltpu.VMEM((PCH, CHUNK), jnp.int32),
            pltpu.VMEM((PCH, CHUNK), jnp.int32),
            pltpu.VMEM((CHUNK, D), _f32),
            pltpu.VMEM((CHUNK, D), _f32),
            pltpu.SemaphoreType.DMA,
            pltpu.SemaphoreType.DMA,
            pltpu.SemaphoreType.DMA,
            pltpu.SemaphoreType.DMA,
            pltpu.VMEM_SHARED((NPAD, D), _f32),
        ],
    )


def _sc_segsum(gidx, sidx, table):
    return _make_sc_segsum()(gidx, sidx, table)


BR = 1000
GRID = N_NODES // BR


def _pack_cols(p):
    lo = jax.lax.bitcast_convert_type(p[:, :DW].astype(jnp.bfloat16),
                                      jnp.uint16).astype(jnp.uint32)
    hi = jax.lax.bitcast_convert_type(p[:, DW:].astype(jnp.bfloat16),
                                      jnp.uint16).astype(jnp.uint32)
    return jax.lax.bitcast_convert_type(lo | (hi << 16), _f32)


def _embed_body(x_ref, t_ref, params_ref, Wsp_ref, bsp_ref, tt_ref,
                Wpp_ref, bpp_ref, W0_ref, hs_ref, hr_ref, ps_ref):
    hs = jnp.maximum(jnp.log1p(x_ref[...]) * Wsp_ref[...] + bsp_ref[...], 0.0)
    hs_ref[...] = hs
    ps_ref[...] = jnp.dot(hs, W0_ref[...], preferred_element_type=_f32)
    onehot = (t_ref[...] == lax.broadcasted_iota(jnp.int32, (BR, N_TYPES), 1)
              ).astype(_f32)
    hr = (jnp.dot(onehot, tt_ref[...], preferred_element_type=_f32)
          + jnp.dot(params_ref[...], Wpp_ref[...], preferred_element_type=_f32)
          + bpp_ref[...])
    hr_ref[...] = jnp.maximum(hr, 0.0)


def _embed(x2, t2, params, Wsp, bsp2, tt, Wpp, bpp2, W0):
    full = lambda shape: pl.BlockSpec(shape, lambda i: (0,) * len(shape))
    rows = lambda w: pl.BlockSpec((BR, w), lambda i: (i, 0))
    return pl.pallas_call(
        _embed_body,
        grid=(GRID,),
        in_specs=[rows(1), rows(1), rows(4), full((1, D)), full((1, D)),
                  full((N_TYPES, D)), full((4, D)), full((1, D)),
                  full((D, D))],
        out_specs=[rows(D), rows(D), rows(D)],
        out_shape=[jax.ShapeDtypeStruct((N_NODES, D), _f32)] * 3,
    )(x2, t2, params, Wsp, bsp2, tt, Wpp, bpp2, W0)


def _update_body(with_p, with_sum, *refs):
    it = iter(refs)
    h_ref, p0_ref, p1_ref = next(it), next(it), next(it)
    W_ref = next(it) if with_p else None
    b_ref = next(it)
    out_ref = next(it)
    pn_ref = next(it) if with_p else None
    sum_ref = next(it) if with_sum else None
    h = jnp.maximum(h_ref[...] + p0_ref[0] + p1_ref[0] + b_ref[...], 0.0)
    out_ref[...] = h
    if with_p:
        pn_ref[...] = jnp.dot(h, W_ref[...], preferred_element_type=_f32)
    if with_sum:
        @pl.when(pl.program_id(0) == 0)
        def _():
            sum_ref[...] = jnp.zeros_like(sum_ref)
        sum_ref[...] += jnp.sum(h, axis=0, keepdims=True)


def _update(h, parts, W_next, b2, with_sum):
    with_p = W_next is not None
    rows = pl.BlockSpec((BR, D), lambda i: (i, 0))
    full = lambda shape: pl.BlockSpec(shape, lambda i: (0, 0))
    in_specs = [rows,
                pl.BlockSpec((1, BR, D), lambda i: (0, i, 0)),
                pl.BlockSpec((1, BR, D), lambda i: (1, i, 0))]
    args = [h, parts, parts]
    if with_p:
        in_specs.append(full((D, D)))
        args.append(W_next)
    in_specs.append(full((1, D)))
    args.append(b2)
    out_specs = [rows]
    out_shape = [jax.ShapeDtypeStruct((N_NODES, D), _f32)]
    if with_p:
        out_specs.append(rows)
        out_shape.append(jax.ShapeDtypeStruct((N_NODES, D), _f32))
    if with_sum:
        out_specs.append(full((1, D)))
        out_shape.append(jax.ShapeDtypeStruct((1, D), _f32))
    res = pl.pallas_call(
        functools.partial(_update_body, with_p, with_sum),
        grid=(GRID,),
        in_specs=in_specs,
        out_specs=out_specs,
        out_shape=out_shape,
    )(*args)
    return res


def kernel(initial_state, propensity_params, W_sp, b_sp, type_table, W_pp,
           b_pp, W_s2r, b_s2r, W_r2s, b_r2s, propensity_types, edge_index):
    x2 = initial_state.reshape(N_NODES, 1)
    t2 = propensity_types.reshape(N_NODES, 1)
    src = edge_index[0].reshape(NW * NCH, CHUNK)
    dst = edge_index[1].reshape(NW * NCH, CHUNK)
    bsp2 = b_sp.reshape(1, D)
    bpp2 = b_pp.reshape(1, D)

    h_s, h_r, p_s = _embed(x2, t2, propensity_params, W_sp, bsp2, type_table,
                           W_pp, bpp2, W_s2r[0])

    sum_r = sum_s = None
    for l in range(N_LAYERS):
        last = l == N_LAYERS - 1
        parts = _sc_segsum(src, dst, p_s)
        res = _update(h_r, parts, W_r2s[l], b_s2r[l].reshape(1, D), last)
        if last:
            h_r, p_r, sum_r = res
        else:
            h_r, p_r = res
        parts = _sc_segsum(dst, src, p_r)
        res = _update(h_s, parts,
                      None if last else W_s2r[l + 1],
                      b_r2s[l].reshape(1, D), last)
        if last:
            h_s, sum_s = res
        else:
            h_s, p_s = res

    context = jnp.concatenate(
        [sum_s[0] * (1.0 / N_NODES), sum_r[0] * (1.0 / N_NODES)], axis=-1)
    return (h_s, h_r, context)


# speedup vs baseline: 1.7601x; 1.0211x over previous
import functools

import jax
import jax.numpy as jnp
from jax import lax
from jax.experimental import pallas as pl
from jax.experimental.pallas import tpu as pltpu
from jax.experimental.pallas import tpu_sc as plsc

N_NODES = 10000
N_EDGES = 320000
D = 128
N_LAYERS = 3
N_TYPES = 8

NC = 2
NSUB = 16
NW = NC * NSUB
EPW = N_EDGES // NW
CHUNK = 125
NCH = EPW // CHUNK
NPAD = 10112
ROWS_PER_TILE = NPAD // NSUB
PCH = 40

_f32 = jnp.float32


def _sc_segsum_body(gidx_hbm, sidx_hbm, table_hbm, out_hbm,
                    gidx_v, sidx_v, rows0, rows1, g0, g1, accum):
    cid = lax.axis_index("c")
    sid = lax.axis_index("s")
    wid = cid * NSUB + sid
    bufs = (rows0, rows1)
    gsems = (g0, g1)

    def _zrow(i, carry):
        for j in range(D // 16):
            rows0[i, pl.ds(j * 16, 16)] = jnp.zeros((16,), _f32)
        return carry
    lax.fori_loop(0, CHUNK, _zrow, 0)
    row_base = sid * ROWS_PER_TILE
    for off, n in ((0, 120), (120, 120), (240, 120), (360, 120), (480, 120),
                   (600, 32)):
        pltpu.sync_copy(rows0.at[pl.ds(0, n)],
                        accum.at[pl.ds(row_base + off, n)])
    plsc.subcore_barrier()

    def _gather_start(c, b):
        pltpu.async_copy(table_hbm.at[gidx_v.at[c]], bufs[b], gsems[b])

    def _gather_wait(c, b):
        pltpu.make_async_copy(table_hbm.at[gidx_v.at[c]], bufs[b],
                              gsems[b]).wait()

    def _step(c, b, first=False, last=False):
        _gather_wait(c, b)
        pltpu.sync_copy(bufs[b], accum.at[sidx_v.at[c]], add=True)
        if not last:
            _gather_start(c + 2, b)

    for p in range(NCH // PCH):
        pltpu.sync_copy(gidx_hbm.at[pl.ds(wid * NCH + p * PCH, PCH)], gidx_v)
        pltpu.sync_copy(sidx_hbm.at[pl.ds(wid * NCH + p * PCH, PCH)], sidx_v)

        _gather_start(0, 0)
        _gather_start(1, 1)
        _step(0, 0, first=True)
        _step(1, 1, first=True)

        def _pair(q, carry):
            c0 = q * 2
            for b in range(2):
                _step(c0 + b, b)
            return carry
        lax.fori_loop(1, PCH // 2 - 1, _pair, 0)

        _step(PCH - 2, 0, last=True)
        _step(PCH - 1, 1, last=True)

    plsc.subcore_barrier()
    pltpu.sync_copy(
        accum.at[pl.ds(row_base, ROWS_PER_TILE)],
        out_hbm.at[cid, pl.ds(row_base, ROWS_PER_TILE)])


@functools.cache
def _make_sc_segsum():
    return pl.kernel(
        _sc_segsum_body,
        out_type=jax.ShapeDtypeStruct((NC, NPAD, D), _f32),
        mesh=plsc.VectorSubcoreMesh(
            core_axis_name="c", subcore_axis_name="s",
            num_cores=NC, num_subcores=NSUB),
        scratch_types=[
            pltpu.VMEM((PCH, CHUNK), jnp.int32),
            pltpu.VMEM((PCH, CHUNK), jnp.int32),
            pltpu.VMEM((CHUNK, D), _f32),
            pltpu.VMEM((CHUNK, D), _f32),
            pltpu.SemaphoreType.DMA,
            pltpu.SemaphoreType.DMA,
            pltpu.VMEM_SHARED((NPAD, D), _f32),
        ],
    )


def _sc_segsum(gidx, sidx, table):
    return _make_sc_segsum()(gidx, sidx, table)


BR = 1000
GRID = N_NODES // BR


def _pack_cols(p):
    lo = jax.lax.bitcast_convert_type(p[:, :DW].astype(jnp.bfloat16),
                                      jnp.uint16).astype(jnp.uint32)
    hi = jax.lax.bitcast_convert_type(p[:, DW:].astype(jnp.bfloat16),
                                      jnp.uint16).astype(jnp.uint32)
    return jax.lax.bitcast_convert_type(lo | (hi << 16), _f32)


def _embed_body(x_ref, t_ref, params_ref, Wsp_ref, bsp_ref, tt_ref,
                Wpp_ref, bpp_ref, W0_ref, hs_ref, hr_ref, ps_ref):
    hs = jnp.maximum(jnp.log1p(x_ref[...]) * Wsp_ref[...] + bsp_ref[...], 0.0)
    hs_ref[...] = hs
    ps_ref[...] = jnp.dot(hs, W0_ref[...], preferred_element_type=_f32)
    onehot = (t_ref[...] == lax.broadcasted_iota(jnp.int32, (BR, N_TYPES), 1)
              ).astype(_f32)
    hr = (jnp.dot(onehot, tt_ref[...], preferred_element_type=_f32)
          + jnp.dot(params_ref[...], Wpp_ref[...], preferred_element_type=_f32)
          + bpp_ref[...])
    hr_ref[...] = jnp.maximum(hr, 0.0)


def _embed(x2, t2, params, Wsp, bsp2, tt, Wpp, bpp2, W0):
    full = lambda shape: pl.BlockSpec(shape, lambda i: (0,) * len(shape))
    rows = lambda w: pl.BlockSpec((BR, w), lambda i: (i, 0))
    return pl.pallas_call(
        _embed_body,
        grid=(GRID,),
        in_specs=[rows(1), rows(1), rows(4), full((1, D)), full((1, D)),
                  full((N_TYPES, D)), full((4, D)), full((1, D)),
                  full((D, D))],
        out_specs=[rows(D), rows(D), rows(D)],
        out_shape=[jax.ShapeDtypeStruct((N_NODES, D), _f32)] * 3,
    )(x2, t2, params, Wsp, bsp2, tt, Wpp, bpp2, W0)


def _update_body(with_p, with_sum, *refs):
    it = iter(refs)
    h_ref, p0_ref, p1_ref = next(it), next(it), next(it)
    W_ref = next(it) if with_p else None
    b_ref = next(it)
    out_ref = next(it)
    pn_ref = next(it) if with_p else None
    sum_ref = next(it) if with_sum else None
    h = jnp.maximum(h_ref[...] + p0_ref[0] + p1_ref[0] + b_ref[...], 0.0)
    out_ref[...] = h
    if with_p:
        pn_ref[...] = jnp.dot(h, W_ref[...], preferred_element_type=_f32)
    if with_sum:
        @pl.when(pl.program_id(0) == 0)
        def _():
            sum_ref[...] = jnp.zeros_like(sum_ref)
        sum_ref[...] += jnp.sum(h, axis=0, keepdims=True)


def _update(h, parts, W_next, b2, with_sum):
    with_p = W_next is not None
    rows = pl.BlockSpec((BR, D), lambda i: (i, 0))
    full = lambda shape: pl.BlockSpec(shape, lambda i: (0, 0))
    in_specs = [rows,
                pl.BlockSpec((1, BR, D), lambda i: (0, i, 0)),
                pl.BlockSpec((1, BR, D), lambda i: (1, i, 0))]
    args = [h, parts, parts]
    if with_p:
        in_specs.append(full((D, D)))
        args.append(W_next)
    in_specs.append(full((1, D)))
    args.append(b2)
    out_specs = [rows]
    out_shape = [jax.ShapeDtypeStruct((N_NODES, D), _f32)]
    if with_p:
        out_specs.append(rows)
        out_shape.append(jax.ShapeDtypeStruct((N_NODES, D), _f32))
    if with_sum:
        out_specs.append(full((1, D)))
        out_shape.append(jax.ShapeDtypeStruct((1, D), _f32))
    res = pl.pallas_call(
        functools.partial(_update_body, with_p, with_sum),
        grid=(GRID,),
        in_specs=in_specs,
        out_specs=out_specs,
        out_shape=out_shape,
    )(*args)
    return res


def kernel(initial_state, propensity_params, W_sp, b_sp, type_table, W_pp,
           b_pp, W_s2r, b_s2r, W_r2s, b_r2s, propensity_types, edge_index):
    x2 = initial_state.reshape(N_NODES, 1)
    t2 = propensity_types.reshape(N_NODES, 1)
    src = edge_index[0].reshape(NW * NCH, CHUNK)
    dst = edge_index[1].reshape(NW * NCH, CHUNK)
    bsp2 = b_sp.reshape(1, D)
    bpp2 = b_pp.reshape(1, D)

    h_s, h_r, p_s = _embed(x2, t2, propensity_params, W_sp, bsp2, type_table,
                           W_pp, bpp2, W_s2r[0])

    sum_r = sum_s = None
    for l in range(N_LAYERS):
        last = l == N_LAYERS - 1
        parts = _sc_segsum(src, dst, p_s)
        res = _update(h_r, parts, W_r2s[l], b_s2r[l].reshape(1, D), last)
        if last:
            h_r, p_r, sum_r = res
        else:
            h_r, p_r = res
        parts = _sc_segsum(dst, src, p_r)
        res = _update(h_s, parts,
                      None if last else W_s2r[l + 1],
                      b_r2s[l].reshape(1, D), last)
        if last:
            h_s, sum_s = res
        else:
            h_s, p_s = res

    context = jnp.concatenate(
        [sum_s[0] * (1.0 / N_NODES), sum_r[0] * (1.0 / N_NODES)], axis=-1)
    return (h_s, h_r, context)


# speedup vs baseline: 1.7973x; 1.0211x over previous
import functools

import jax
import jax.numpy as jnp
from jax import lax
from jax.experimental import pallas as pl
from jax.experimental.pallas import tpu as pltpu
from jax.experimental.pallas import tpu_sc as plsc

N_NODES = 10000
N_EDGES = 320000
D = 128
N_LAYERS = 3
N_TYPES = 8

NC = 2
NSUB = 16
NW = NC * NSUB
EPW = N_EDGES // NW
CHUNK = 125
NCH = EPW // CHUNK
NPAD = 10112
ROWS_PER_TILE = NPAD // NSUB
PCH = 40

_f32 = jnp.float32


def _sc_segsum_body(gidx_hbm, sidx_hbm, table_hbm, out_hbm,
                    gidx_v, sidx_v, rows0, rows1, g0, g1, accum):
    cid = lax.axis_index("c")
    sid = lax.axis_index("s")
    wid = cid * NSUB + sid
    bufs = (rows0, rows1)
    gsems = (g0, g1)

    def _zrow(i, carry):
        for j in range(D // 16):
            rows0[i, pl.ds(j * 16, 16)] = jnp.zeros((16,), _f32)
        return carry
    lax.fori_loop(0, CHUNK, _zrow, 0)
    row_base = sid * ROWS_PER_TILE
    for off, n in ((0, 120), (120, 120), (240, 120), (360, 120), (480, 120),
                   (600, 32)):
        pltpu.sync_copy(rows0.at[pl.ds(0, n)],
                        accum.at[pl.ds(row_base + off, n)])
    plsc.subcore_barrier()

    def _gather_start(c, b):
        pltpu.async_copy(table_hbm.at[gidx_v.at[c]], bufs[b], gsems[b])

    def _gather_wait(c, b):
        pltpu.make_async_copy(table_hbm.at[gidx_v.at[c]], bufs[b],
                              gsems[b]).wait()

    def _step(c, b, first=False, last=False):
        _gather_wait(c, b)
        pltpu.sync_copy(bufs[b], accum.at[sidx_v.at[c]], add=True)
        if not last:
            _gather_start(c + 2, b)

    for p in range(NCH // PCH):
        pltpu.sync_copy(gidx_hbm.at[pl.ds(wid * NCH + p * PCH, PCH)], gidx_v)
        pltpu.sync_copy(sidx_hbm.at[pl.ds(wid * NCH + p * PCH, PCH)], sidx_v)

        _gather_start(0, 0)
        _gather_start(1, 1)
        _step(0, 0, first=True)
        _step(1, 1, first=True)

        def _pair(q, carry):
            c0 = q * 2
            for b in range(2):
                _step(c0 + b, b)
            return carry
        lax.fori_loop(1, PCH // 2 - 1, _pair, 0)

        _step(PCH - 2, 0, last=True)
        _step(PCH - 1, 1, last=True)

    plsc.subcore_barrier()
    pltpu.sync_copy(
        accum.at[pl.ds(row_base, ROWS_PER_TILE)],
        out_hbm.at[cid, pl.ds(row_base, ROWS_PER_TILE)])


@functools.cache
def _make_sc_segsum():
    return pl.kernel(
        _sc_segsum_body,
        out_type=jax.ShapeDtypeStruct((NC, NPAD, D), _f32),
        mesh=plsc.VectorSubcoreMesh(
            core_axis_name="c", subcore_axis_name="s",
            num_cores=NC, num_subcores=NSUB),
        scratch_types=[
            pltpu.VMEM((PCH, CHUNK), jnp.int32),
            pltpu.VMEM((PCH, CHUNK), jnp.int32),
            pltpu.VMEM((CHUNK, D), _f32),
            pltpu.VMEM((CHUNK, D), _f32),
            pltpu.SemaphoreType.DMA,
            pltpu.SemaphoreType.DMA,
            pltpu.VMEM_SHARED((NPAD, D), _f32),
        ],
    )


def _sc_segsum(gidx, sidx, table):
    return _make_sc_segsum()(gidx, sidx, table)


BR = 2000
GRID = N_NODES // BR


def _embed_body(x_ref, t_ref, params_ref, Wsp_ref, bsp_ref, tt_ref,
                Wpp_ref, bpp_ref, W0_ref, hs_ref, hr_ref, ps_ref):
    hs = jnp.maximum(jnp.log1p(x_ref[...]) * Wsp_ref[...] + bsp_ref[...], 0.0)
    hs_ref[...] = hs
    ps_ref[...] = jnp.dot(hs, W0_ref[...], preferred_element_type=_f32)
    onehot = (t_ref[...] == lax.broadcasted_iota(jnp.int32, (BR, N_TYPES), 1)
              ).astype(_f32)
    hr = (jnp.dot(onehot, tt_ref[...], preferred_element_type=_f32)
          + jnp.dot(params_ref[...], Wpp_ref[...], preferred_element_type=_f32)
          + bpp_ref[...])
    hr_ref[...] = jnp.maximum(hr, 0.0)


def _embed(x2, t2, params, Wsp, bsp2, tt, Wpp, bpp2, W0):
    full = lambda shape: pl.BlockSpec(shape, lambda i: (0,) * len(shape))
    rows = lambda w: pl.BlockSpec((BR, w), lambda i: (i, 0))
    return pl.pallas_call(
        _embed_body,
        grid=(GRID,),
        in_specs=[rows(1), rows(1), rows(4), full((1, D)), full((1, D)),
                  full((N_TYPES, D)), full((4, D)), full((1, D)),
                  full((D, D))],
        out_specs=[rows(D), rows(D), rows(D)],
        out_shape=[jax.ShapeDtypeStruct((N_NODES, D), _f32)] * 3,
    )(x2, t2, params, Wsp, bsp2, tt, Wpp, bpp2, W0)


def _update_body(with_p, with_sum, *refs):
    it = iter(refs)
    h_ref, p0_ref, p1_ref = next(it), next(it), next(it)
    W_ref = next(it) if with_p else None
    b_ref = next(it)
    out_ref = next(it)
    pn_ref = next(it) if with_p else None
    sum_ref = next(it) if with_sum else None
    h = jnp.maximum(h_ref[...] + p0_ref[0] + p1_ref[0] + b_ref[...], 0.0)
    out_ref[...] = h
    if with_p:
        pn_ref[...] = jnp.dot(h, W_ref[...], preferred_element_type=_f32)
    if with_sum:
        @pl.when(pl.program_id(0) == 0)
        def _():
            sum_ref[...] = jnp.zeros_like(sum_ref)
        sum_ref[...] += jnp.sum(h, axis=0, keepdims=True)


def _update(h, parts, W_next, b2, with_sum):
    with_p = W_next is not None
    rows = pl.BlockSpec((BR, D), lambda i: (i, 0))
    full = lambda shape: pl.BlockSpec(shape, lambda i: (0, 0))
    in_specs = [rows,
                pl.BlockSpec((1, BR, D), lambda i: (0, i, 0)),
                pl.BlockSpec((1, BR, D), lambda i: (1, i, 0))]
    args = [h, parts, parts]
    if with_p:
        in_specs.append(full((D, D)))
        args.append(W_next)
    in_specs.append(full((1, D)))
    args.append(b2)
    out_specs = [rows]
    out_shape = [jax.ShapeDtypeStruct((N_NODES, D), _f32)]
    if with_p:
        out_specs.append(rows)
        out_shape.append(jax.ShapeDtypeStruct((N_NODES, D), _f32))
    if with_sum:
        out_specs.append(full((1, D)))
        out_shape.append(jax.ShapeDtypeStruct((1, D), _f32))
    res = pl.pallas_call(
        functools.partial(_update_body, with_p, with_sum),
        grid=(GRID,),
        in_specs=in_specs,
        out_specs=out_specs,
        out_shape=out_shape,
    )(*args)
    return res


def kernel(initial_state, propensity_params, W_sp, b_sp, type_table, W_pp,
           b_pp, W_s2r, b_s2r, W_r2s, b_r2s, propensity_types, edge_index):
    x2 = initial_state.reshape(N_NODES, 1)
    t2 = propensity_types.reshape(N_NODES, 1)
    src = edge_index[0].reshape(NW * NCH, CHUNK)
    dst = edge_index[1].reshape(NW * NCH, CHUNK)
    bsp2 = b_sp.reshape(1, D)
    bpp2 = b_pp.reshape(1, D)

    h_s, h_r, p_s = _embed(x2, t2, propensity_params, W_sp, bsp2, type_table,
                           W_pp, bpp2, W_s2r[0])

    sum_r = sum_s = None
    for l in range(N_LAYERS):
        last = l == N_LAYERS - 1
        parts = _sc_segsum(src, dst, p_s)
        res = _update(h_r, parts, W_r2s[l], b_s2r[l].reshape(1, D), last)
        if last:
            h_r, p_r, sum_r = res
        else:
            h_r, p_r = res
        parts = _sc_segsum(dst, src, p_r)
        res = _update(h_s, parts,
                      None if last else W_s2r[l + 1],
                      b_r2s[l].reshape(1, D), last)
        if last:
            h_s, sum_s = res
        else:
            h_s, p_s = res

    context = jnp.concatenate(
        [sum_s[0] * (1.0 / N_NODES), sum_r[0] * (1.0 / N_NODES)], axis=-1)
    return (h_s, h_r, context)


# speedup vs baseline: 1.8243x; 1.0151x over previous
import functools

import jax
import jax.numpy as jnp
from jax import lax
from jax.experimental import pallas as pl
from jax.experimental.pallas import tpu as pltpu
from jax.experimental.pallas import tpu_sc as plsc

N_NODES = 10000
N_EDGES = 320000
D = 128
N_LAYERS = 3
N_TYPES = 8

NC = 2
NSUB = 16
NW = NC * NSUB
EPW = N_EDGES // NW
CHUNK = 125
NCH = EPW // CHUNK
NPAD = 10112
ROWS_PER_TILE = NPAD // NSUB
PCH = 40

_f32 = jnp.float32


def _sc_segsum_body(flip, edges_hbm, table_hbm, out_hbm,
                    gidx_v, sidx_v, rows0, rows1, g0, g1, accum):
    gslot, sslot = (1, 0) if flip else (0, 1)
    cid = lax.axis_index("c")
    sid = lax.axis_index("s")
    wid = cid * NSUB + sid
    bufs = (rows0, rows1)
    gsems = (g0, g1)

    def _zrow(i, carry):
        for j in range(D // 16):
            rows0[i, pl.ds(j * 16, 16)] = jnp.zeros((16,), _f32)
        return carry
    lax.fori_loop(0, CHUNK, _zrow, 0)
    row_base = sid * ROWS_PER_TILE
    for off, n in ((0, 120), (120, 120), (240, 120), (360, 120), (480, 120),
                   (600, 32)):
        pltpu.sync_copy(rows0.at[pl.ds(0, n)],
                        accum.at[pl.ds(row_base + off, n)])
    plsc.subcore_barrier()

    def _gather_start(c, b):
        pltpu.async_copy(table_hbm.at[gidx_v.at[c]], bufs[b], gsems[b])

    def _gather_wait(c, b):
        pltpu.make_async_copy(table_hbm.at[gidx_v.at[c]], bufs[b],
                              gsems[b]).wait()

    def _step(c, b, first=False, last=False):
        _gather_wait(c, b)
        pltpu.sync_copy(bufs[b], accum.at[sidx_v.at[c]], add=True)
        if not last:
            _gather_start(c + 2, b)

    for p in range(NCH // PCH):
        pltpu.sync_copy(
            edges_hbm.at[gslot, pl.ds(wid * NCH + p * PCH, PCH)], gidx_v)
        pltpu.sync_copy(
            edges_hbm.at[sslot, pl.ds(wid * NCH + p * PCH, PCH)], sidx_v)

        _gather_start(0, 0)
        _gather_start(1, 1)
        _step(0, 0, first=True)
        _step(1, 1, first=True)

        def _pair(q, carry):
            c0 = q * 2
            for b in range(2):
                _step(c0 + b, b)
            return carry
        lax.fori_loop(1, PCH // 2 - 1, _pair, 0)

        _step(PCH - 2, 0, last=True)
        _step(PCH - 1, 1, last=True)

    plsc.subcore_barrier()
    pltpu.sync_copy(
        accum.at[pl.ds(row_base, ROWS_PER_TILE)],
        out_hbm.at[cid, pl.ds(row_base, ROWS_PER_TILE)])


@functools.cache
def _make_sc_segsum(flip):
    return pl.kernel(
        functools.partial(_sc_segsum_body, flip),
        out_type=jax.ShapeDtypeStruct((NC, NPAD, D), _f32),
        mesh=plsc.VectorSubcoreMesh(
            core_axis_name="c", subcore_axis_name="s",
            num_cores=NC, num_subcores=NSUB),
        scratch_types=[
            pltpu.VMEM((PCH, CHUNK), jnp.int32),
            pltpu.VMEM((PCH, CHUNK), jnp.int32),
            pltpu.VMEM((CHUNK, D), _f32),
            pltpu.VMEM((CHUNK, D), _f32),
            pltpu.SemaphoreType.DMA,
            pltpu.SemaphoreType.DMA,
            pltpu.VMEM_SHARED((NPAD, D), _f32),
        ],
    )


def _sc_segsum(edges, flip, table):
    return _make_sc_segsum(flip)(edges, table)


BR = 2000
GRID = N_NODES // BR


def _embed_body(x_ref, t_ref, params_ref, Wsp_ref, bsp_ref, tt_ref,
                Wpp_ref, bpp_ref, W0_ref, hs_ref, hr_ref, ps_ref):
    hs = jnp.maximum(jnp.log1p(x_ref[...]) * Wsp_ref[...] + bsp_ref[...], 0.0)
    hs_ref[...] = hs
    ps_ref[...] = jnp.dot(hs, W0_ref[0], preferred_element_type=_f32)
    onehot = (t_ref[...] == lax.broadcasted_iota(jnp.int32, (BR, N_TYPES), 1)
              ).astype(_f32)
    hr = (jnp.dot(onehot, tt_ref[...], preferred_element_type=_f32)
          + jnp.dot(params_ref[...], Wpp_ref[...], preferred_element_type=_f32)
          + bpp_ref[...])
    hr_ref[...] = jnp.maximum(hr, 0.0)


def _embed(x2, t2, params, Wsp, bsp, tt, Wpp, bpp, W_s2r):
    full = lambda shape: pl.BlockSpec(shape, lambda i: (0,) * len(shape))
    rows = lambda w: pl.BlockSpec((BR, w), lambda i: (i, 0))
    return pl.pallas_call(
        _embed_body,
        grid=(GRID,),
        in_specs=[rows(1), rows(1), rows(4), full((1, D)), full((D,)),
                  full((N_TYPES, D)), full((4, D)), full((D,)),
                  full((1, D, D))],
        out_specs=[rows(D), rows(D), rows(D)],
        out_shape=[jax.ShapeDtypeStruct((N_NODES, D), _f32)] * 3,
    )(x2, t2, params, Wsp, bsp, tt, Wpp, bpp, W_s2r)


def _update_body(with_p, with_sum, *refs):
    it = iter(refs)
    h_ref, p0_ref, p1_ref = next(it), next(it), next(it)
    W_ref = next(it) if with_p else None
    b_ref = next(it)
    out_ref = next(it)
    pn_ref = next(it) if with_p else None
    sum_ref = next(it) if with_sum else None
    h = jnp.maximum(h_ref[...] + p0_ref[0] + p1_ref[0] + b_ref[0], 0.0)
    out_ref[...] = h
    if with_p:
        pn_ref[...] = jnp.dot(h, W_ref[0], preferred_element_type=_f32)
    if with_sum:
        @pl.when(pl.program_id(0) == 0)
        def _():
            sum_ref[...] = jnp.zeros_like(sum_ref)
        sum_ref[...] += jnp.sum(h, axis=0, keepdims=True)


def _update(h, parts, W_all, wl, b_all, bl, with_sum):
    with_p = W_all is not None
    rows = pl.BlockSpec((BR, D), lambda i: (i, 0))
    full = lambda shape: pl.BlockSpec(shape, lambda i: (0, 0))
    in_specs = [rows,
                pl.BlockSpec((1, BR, D), lambda i: (0, i, 0)),
                pl.BlockSpec((1, BR, D), lambda i: (1, i, 0))]
    args = [h, parts, parts]
    if with_p:
        in_specs.append(pl.BlockSpec((1, D, D), lambda i: (wl, 0, 0)))
        args.append(W_all)
    in_specs.append(pl.BlockSpec((1, 1, D), lambda i: (bl, 0, 0)))
    args.append(b_all)
    out_specs = [rows]
    out_shape = [jax.ShapeDtypeStruct((N_NODES, D), _f32)]
    if with_p:
        out_specs.append(rows)
        out_shape.append(jax.ShapeDtypeStruct((N_NODES, D), _f32))
    if with_sum:
        out_specs.append(full((1, D)))
        out_shape.append(jax.ShapeDtypeStruct((1, D), _f32))
    res = pl.pallas_call(
        functools.partial(_update_body, with_p, with_sum),
        grid=(GRID,),
        in_specs=in_specs,
        out_specs=out_specs,
        out_shape=out_shape,
    )(*args)
    return res


def kernel(initial_state, propensity_params, W_sp, b_sp, type_table, W_pp,
           b_pp, W_s2r, b_s2r, W_r2s, b_r2s, propensity_types, edge_index):
    x2 = initial_state.reshape(N_NODES, 1)
    t2 = propensity_types.reshape(N_NODES, 1)
    edges = edge_index.reshape(2, NW * NCH, CHUNK)
    b_s2r3 = b_s2r.reshape(N_LAYERS, 1, D)
    b_r2s3 = b_r2s.reshape(N_LAYERS, 1, D)

    h_s, h_r, p_s = _embed(x2, t2, propensity_params, W_sp, b_sp, type_table,
                           W_pp, b_pp, W_s2r)

    sum_r = sum_s = None
    for l in range(N_LAYERS):
        last = l == N_LAYERS - 1
        parts = _sc_segsum(edges, False, p_s)
        res = _update(h_r, parts, W_r2s, l, b_s2r3, l, last)
        if last:
            h_r, p_r, sum_r = res
        else:
            h_r, p_r = res
        parts = _sc_segsum(edges, True, p_r)
        res = _update(h_s, parts,
                      None if last else W_s2r, l + 1,
                      b_r2s3, l, last)
        if last:
            h_s, sum_s = res
        else:
            h_s, p_s = res

    context = jnp.concatenate(
        [sum_s[0] * (1.0 / N_NODES), sum_r[0] * (1.0 / N_NODES)], axis=-1)
    return (h_s, h_r, context)
